# baseline (speedup 1.0000x reference)
import functools

import jax
import jax.numpy as jnp
from jax import lax
from jax.experimental import pallas as pl
from jax.experimental.pallas import tpu as pltpu
from jax.experimental.pallas import tpu_sc as plsc

N = 10000
E = 320000
IN_DIM = 128
HID = 64

NC = 2
NS = 16
NW = NC * NS
EPT = E // NW
CW = 125
CR = 80
EPTP = CR * CW
ACC_ROWS = N + 16
RPT = N // NS
ZROWS = 125
DEG_W = 16
NB = 5

BLK = 2000
GRID = N // BLK


def _fill_vmem(ref, rows, width, value):
    def row(i, _):
        def col(j, _):
            ref[i, pl.ds(j * 16, 16)] = jnp.full((16,), value, jnp.float32)
            return 0
        return lax.fori_loop(0, width // 16, col, 0)
    lax.fori_loop(0, rows, row, 0)


def _init_acc(zbuf_v, acc_sh, sid, width):
    _fill_vmem(zbuf_v, ZROWS, width, 0.0)
    for k in range(RPT // ZROWS):
        pltpu.sync_copy(zbuf_v, acc_sh.at[pl.ds(sid * RPT + k * ZROWS, ZROWS)])


def _sc_degree(dst3d):
    mesh = plsc.VectorSubcoreMesh(core_axis_name="c", subcore_axis_name="s")

    @functools.partial(
        pl.kernel,
        mesh=mesh,
        out_type=jax.ShapeDtypeStruct((NW, RPT, DEG_W), jnp.float32),
        compiler_params=pltpu.CompilerParams(use_tc_tiling_on_sc=False),
        scratch_types=[
            pltpu.VMEM((CR, CW), jnp.int32),
            pltpu.VMEM((CW, DEG_W), jnp.float32),
            pltpu.VMEM((ZROWS, DEG_W), jnp.float32),
            pltpu.VMEM_SHARED((ACC_ROWS, DEG_W), jnp.float32),
            pltpu.SemaphoreType.DMA,
            pltpu.SemaphoreType.DMA,
        ],
    )
    def k(dst_hbm, out_hbm, dst_v, ones_v, zbuf_v, acc_sh, sem, isem):
        cid = lax.axis_index("c")
        sid = lax.axis_index("s")
        wid = cid * NS + sid

        pltpu.async_copy(dst_hbm.at[wid], dst_v, isem)
        _fill_vmem(ones_v, CW, DEG_W, 1.0)
        _init_acc(zbuf_v, acc_sh, sid, DEG_W)
        pltpu.make_async_copy(dst_hbm.at[wid], dst_v, isem).wait()
        plsc.subcore_barrier()

        def body(j, _):
            pltpu.async_copy(ones_v, acc_sh.at[dst_v.at[j]], sem, add=True)

            @pl.when(j >= 4)
            def _():
                pltpu.make_async_copy(ones_v, acc_sh.at[dst_v.at[j]],
                                      sem).wait()
            return 0
        lax.fori_loop(0, CR, body, 0)
        for j in range(4):
            pltpu.make_async_copy(ones_v, acc_sh.at[dst_v.at[j]], sem).wait()

        plsc.subcore_barrier()
        pltpu.sync_copy(acc_sh.at[pl.ds(sid * RPT, RPT)], out_hbm.at[wid])

    return k(dst3d)


def _sc_scatter(src3d, dst3d, y):
    mesh = plsc.VectorSubcoreMesh(core_axis_name="c", subcore_axis_name="s")

    @functools.partial(
        pl.kernel,
        mesh=mesh,
        out_type=jax.ShapeDtypeStruct((NW, RPT, HID), jnp.float32),
        compiler_params=pltpu.CompilerParams(use_tc_tiling_on_sc=False),
        scratch_types=[
            pltpu.VMEM((CR, CW), jnp.int32),
            pltpu.VMEM((CR, CW), jnp.int32),
        ]
        + [pltpu.VMEM((CW, HID), jnp.float32) for _ in range(NB)]
        + [
            pltpu.VMEM((ZROWS, HID), jnp.float32),
            pltpu.VMEM_SHARED((ACC_ROWS, HID), jnp.float32),
        ]
        + [pltpu.SemaphoreType.DMA for _ in range(2 * NB + 1)],
    )
    def k(src_hbm, dst_hbm, y_hbm, out_hbm, src_v, dst_v, *rest):
        rows = rest[:NB]
        zbuf_v = rest[NB]
        acc_sh = rest[NB + 1]
        gsem = rest[NB + 2:NB + 2 + NB]
        ssem = rest[NB + 2 + NB:NB + 2 + 2 * NB]
        isem = rest[NB + 2 + 2 * NB]

        cid = lax.axis_index("c")
        sid = lax.axis_index("s")
        wid = cid * NS + sid

        pltpu.async_copy(src_hbm.at[wid], src_v, isem)
        pltpu.async_copy(dst_hbm.at[wid], dst_v, isem)
        _fill_vmem(zbuf_v, ZROWS, HID, 0.0)
        pltpu.make_async_copy(src_hbm.at[wid], src_v, isem).wait()
        pltpu.make_async_copy(dst_hbm.at[wid], dst_v, isem).wait()

        for b in range(NB):
            pltpu.async_copy(y_hbm.at[src_v.at[b]], rows[b], gsem[b])
        for kk in range(RPT // ZROWS):
            pltpu.sync_copy(zbuf_v,
                            acc_sh.at[pl.ds(sid * RPT + kk * ZROWS, ZROWS)])
        plsc.subcore_barrier()

        n_outer = CR // NB

        def outer(i, _):
            base = i * NB
            for b in range(NB):
                j = base + b
                pltpu.make_async_copy(y_hbm.at[src_v.at[j]], rows[b],
                                      gsem[b]).wait()
                pltpu.async_copy(rows[b], acc_sh.at[dst_v.at[j]], ssem[b],
                                 add=True)
            for b in range(NB):
                j = base + b

                @pl.when(i < n_outer - 1)
                def _():
                    pltpu.make_async_copy(rows[b], acc_sh.at[dst_v.at[j]],
                                          ssem[b]).wait()
                    pltpu.async_copy(y_hbm.at[src_v.at[j + NB]], rows[b],
                                     gsem[b])
            return 0

        lax.fori_loop(0, n_outer, outer, 0)
        for b in range(NB):
            j = CR - NB + b
            pltpu.make_async_copy(rows[b], acc_sh.at[dst_v.at[j]],
                                  ssem[b]).wait()

        plsc.subcore_barrier()
        pltpu.sync_copy(acc_sh.at[pl.ds(sid * RPT, RPT)], out_hbm.at[wid])

    return k(src3d, dst3d, y)


def _dis_block(d0_ref, d1_ref):
    deg = d0_ref[:, 0:1] + d1_ref[:, 0:1] + 1.0
    return lax.rsqrt(deg)


def _tc_first(x, W0, degparts):
    def body(x_ref, w_ref, d0_ref, d1_ref, y_ref):
        dis = _dis_block(d0_ref, d1_ref)
        xw = jnp.dot(x_ref[...], w_ref[...], preferred_element_type=jnp.float32)
        y_ref[...] = xw * dis

    return pl.pallas_call(
        body,
        grid=(GRID,),
        in_specs=[
            pl.BlockSpec((BLK, IN_DIM), lambda i: (i, 0)),
            pl.BlockSpec((IN_DIM, HID), lambda i: (0, 0)),
            pl.BlockSpec((BLK, DEG_W), lambda i: (i, 0)),
            pl.BlockSpec((BLK, DEG_W), lambda i: (i + GRID, 0)),
        ],
        out_specs=pl.BlockSpec((BLK, HID), lambda i: (i, 0)),
        out_shape=jax.ShapeDtypeStruct((N, HID), jnp.float32),
    )(x, W0, degparts, degparts)


def _tc_mid(zp, y, degparts, b, Wn):
    def body(z0_ref, z1_ref, y_ref, d0_ref, d1_ref, b_ref, w_ref, o_ref):
        dis = _dis_block(d0_ref, d1_ref)
        z = z0_ref[...] + z1_ref[...] + y_ref[...]
        h = jnp.maximum(z * dis + b_ref[...], 0.0)
        o_ref[...] = jnp.dot(h, w_ref[...], preferred_element_type=jnp.float32) * dis

    return pl.pallas_call(
        body,
        grid=(GRID,),
        in_specs=[
            pl.BlockSpec((BLK, HID), lambda i: (i, 0)),
            pl.BlockSpec((BLK, HID), lambda i: (i + GRID, 0)),
            pl.BlockSpec((BLK, HID), lambda i: (i, 0)),
            pl.BlockSpec((BLK, DEG_W), lambda i: (i, 0)),
            pl.BlockSpec((BLK, DEG_W), lambda i: (i + GRID, 0)),
            pl.BlockSpec((1, HID), lambda i: (0, 0)),
            pl.BlockSpec((HID, HID), lambda i: (0, 0)),
        ],
        out_specs=pl.BlockSpec((BLK, HID), lambda i: (i, 0)),
        out_shape=jax.ShapeDtypeStruct((N, HID), jnp.float32),
    )(zp, zp, y, degparts, degparts, b, Wn)


def _tc_final(zp, y, degparts, b):
    def body(z0_ref, z1_ref, y_ref, d0_ref, d1_ref, b_ref, o_ref):
        dis = _dis_block(d0_ref, d1_ref)
        z = z0_ref[...] + z1_ref[...] + y_ref[...]
        o_ref[...] = z * dis + b_ref[...]

    return pl.pallas_call(
        body,
        grid=(GRID,),
        in_specs=[
            pl.BlockSpec((BLK, HID), lambda i: (i, 0)),
            pl.BlockSpec((BLK, HID), lambda i: (i + GRID, 0)),
            pl.BlockSpec((BLK, HID), lambda i: (i, 0)),
            pl.BlockSpec((BLK, DEG_W), lambda i: (i, 0)),
            pl.BlockSpec((BLK, DEG_W), lambda i: (i + GRID, 0)),
            pl.BlockSpec((1, HID), lambda i: (0, 0)),
        ],
        out_specs=pl.BlockSpec((BLK, HID), lambda i: (i, 0)),
        out_shape=jax.ShapeDtypeStruct((N, HID), jnp.float32),
    )(zp, zp, y, degparts, degparts, b)


def kernel(x, edge_index, W0, b0, W1, b1, W2, b2):
    ei = edge_index.astype(jnp.int32)
    if EPTP > EPT:
        pad_src = jnp.zeros((NW, EPTP - EPT), jnp.int32)
        pad_dst = jnp.broadcast_to(
            N + (jnp.arange(EPTP - EPT, dtype=jnp.int32) % 16),
            (NW, EPTP - EPT))
        src3d = jnp.concatenate([ei[0].reshape(NW, EPT), pad_src], 1)
        dst3d = jnp.concatenate([ei[1].reshape(NW, EPT), pad_dst], 1)
    else:
        src3d, dst3d = ei[0], ei[1]
    src3d = src3d.reshape(NW, CR, CW)
    dst3d = dst3d.reshape(NW, CR, CW)

    degparts = _sc_degree(dst3d).reshape(NW * RPT, DEG_W)
    y = _tc_first(x, W0, degparts)
    zp = _sc_scatter(src3d, dst3d, y).reshape(NW * RPT, HID)
    y = _tc_mid(zp, y, degparts, b0.reshape(1, -1), W1)
    zp = _sc_scatter(src3d, dst3d, y).reshape(NW * RPT, HID)
    y = _tc_mid(zp, y, degparts, b1.reshape(1, -1), W2)
    zp = _sc_scatter(src3d, dst3d, y).reshape(NW * RPT, HID)
    out = _tc_final(zp, y, degparts, b2.reshape(1, -1))
    return out


# speedup vs baseline: 1.0230x; 1.0230x over previous
import functools

import jax
import jax.numpy as jnp
from jax import lax
from jax.experimental import pallas as pl
from jax.experimental.pallas import tpu as pltpu
from jax.experimental.pallas import tpu_sc as plsc

N = 10000
E = 320000
IN_DIM = 128
HID = 64

NC = 2
NS = 16
NW = NC * NS
EPT = E // NW
CW = 125
CR = 80
EPTP = CR * CW
ACC_ROWS = N + 16
RPT = N // NS
ZROWS = 125
DEG_W = 16
NB = 8

BLK = 2000
GRID = N // BLK


def _fill_vmem(ref, rows, width, value):
    def row(i, _):
        def col(j, _):
            ref[i, pl.ds(j * 16, 16)] = jnp.full((16,), value, jnp.float32)
            return 0
        return lax.fori_loop(0, width // 16, col, 0)
    lax.fori_loop(0, rows, row, 0)


def _init_acc(zbuf_v, acc_sh, sid, width):
    _fill_vmem(zbuf_v, ZROWS, width, 0.0)
    for k in range(RPT // ZROWS):
        pltpu.sync_copy(zbuf_v, acc_sh.at[pl.ds(sid * RPT + k * ZROWS, ZROWS)])


def _sc_degree(dst3d):
    mesh = plsc.VectorSubcoreMesh(core_axis_name="c", subcore_axis_name="s")

    @functools.partial(
        pl.kernel,
        mesh=mesh,
        out_type=jax.ShapeDtypeStruct((NW, RPT, DEG_W), jnp.float32),
        compiler_params=pltpu.CompilerParams(use_tc_tiling_on_sc=False),
        scratch_types=[
            pltpu.VMEM((CR, CW), jnp.int32),
            pltpu.VMEM((CW, DEG_W), jnp.float32),
            pltpu.VMEM((ZROWS, DEG_W), jnp.float32),
            pltpu.VMEM_SHARED((ACC_ROWS, DEG_W), jnp.float32),
            pltpu.SemaphoreType.DMA,
            pltpu.SemaphoreType.DMA,
        ],
    )
    def k(dst_hbm, out_hbm, dst_v, ones_v, zbuf_v, acc_sh, sem, isem):
        cid = lax.axis_index("c")
        sid = lax.axis_index("s")
        wid = cid * NS + sid

        pltpu.async_copy(dst_hbm.at[wid], dst_v, isem)
        _fill_vmem(ones_v, CW, DEG_W, 1.0)
        _init_acc(zbuf_v, acc_sh, sid, DEG_W)
        pltpu.make_async_copy(dst_hbm.at[wid], dst_v, isem).wait()
        plsc.subcore_barrier()

        def body(j, _):
            pltpu.async_copy(ones_v, acc_sh.at[dst_v.at[j]], sem, add=True)

            @pl.when(j >= 4)
            def _():
                pltpu.make_async_copy(ones_v, acc_sh.at[dst_v.at[j]],
                                      sem).wait()
            return 0
        lax.fori_loop(0, CR, body, 0)
        for j in range(4):
            pltpu.make_async_copy(ones_v, acc_sh.at[dst_v.at[j]], sem).wait()

        plsc.subcore_barrier()
        pltpu.sync_copy(acc_sh.at[pl.ds(sid * RPT, RPT)], out_hbm.at[wid])

    return k(dst3d)


def _sc_scatter(src3d, dst3d, y):
    mesh = plsc.VectorSubcoreMesh(core_axis_name="c", subcore_axis_name="s")

    @functools.partial(
        pl.kernel,
        mesh=mesh,
        out_type=jax.ShapeDtypeStruct((NW, RPT, HID), jnp.float32),
        compiler_params=pltpu.CompilerParams(use_tc_tiling_on_sc=False),
        scratch_types=[
            pltpu.VMEM((CR, CW), jnp.int32),
            pltpu.VMEM((CR, CW), jnp.int32),
        ]
        + [pltpu.VMEM((CW, HID), jnp.float32) for _ in range(NB)]
        + [pltpu.VMEM_SHARED((ACC_ROWS, HID), jnp.float32)]
        + [pltpu.SemaphoreType.DMA for _ in range(2 * NB + 1)],
    )
    def k(src_hbm, dst_hbm, y_hbm, out_hbm, src_v, dst_v, *rest):
        rows = rest[:NB]
        acc_sh = rest[NB]
        gsem = rest[NB + 1:NB + 1 + NB]
        ssem = rest[NB + 1 + NB:NB + 1 + 2 * NB]
        isem = rest[NB + 1 + 2 * NB]

        cid = lax.axis_index("c")
        sid = lax.axis_index("s")
        wid = cid * NS + sid

        pltpu.async_copy(src_hbm.at[wid], src_v, isem)
        pltpu.async_copy(dst_hbm.at[wid], dst_v, isem)
        _fill_vmem(rows[NB - 1], ZROWS, HID, 0.0)
        pltpu.make_async_copy(src_hbm.at[wid], src_v, isem).wait()
        pltpu.make_async_copy(dst_hbm.at[wid], dst_v, isem).wait()

        for b in range(NB - 1):
            pltpu.async_copy(y_hbm.at[src_v.at[b]], rows[b], gsem[b])
        for kk in range(RPT // ZROWS):
            pltpu.sync_copy(rows[NB - 1],
                            acc_sh.at[pl.ds(sid * RPT + kk * ZROWS, ZROWS)])
        pltpu.async_copy(y_hbm.at[src_v.at[NB - 1]], rows[NB - 1],
                         gsem[NB - 1])
        plsc.subcore_barrier()

        n_outer = CR // NB

        def outer(i, _):
            base = i * NB
            for b in range(NB):
                j = base + b
                pltpu.make_async_copy(y_hbm.at[src_v.at[j]], rows[b],
                                      gsem[b]).wait()
                pltpu.async_copy(rows[b], acc_sh.at[dst_v.at[j]], ssem[b],
                                 add=True)
            for b in range(NB):
                j = base + b

                @pl.when(i < n_outer - 1)
                def _():
                    pltpu.make_async_copy(rows[b], acc_sh.at[dst_v.at[j]],
                                          ssem[b]).wait()
                    pltpu.async_copy(y_hbm.at[src_v.at[j + NB]], rows[b],
                                     gsem[b])
            return 0

        lax.fori_loop(0, n_outer, outer, 0)
        for b in range(NB):
            j = CR - NB + b
            pltpu.make_async_copy(rows[b], acc_sh.at[dst_v.at[j]],
                                  ssem[b]).wait()

        plsc.subcore_barrier()
        pltpu.sync_copy(acc_sh.at[pl.ds(sid * RPT, RPT)], out_hbm.at[wid])

    return k(src3d, dst3d, y)


def _dis_block(d0_ref, d1_ref):
    deg = d0_ref[:, 0:1] + d1_ref[:, 0:1] + 1.0
    return lax.rsqrt(deg)


def _tc_first(x, W0, degparts):
    def body(x_ref, w_ref, d0_ref, d1_ref, y_ref):
        dis = _dis_block(d0_ref, d1_ref)
        xw = jnp.dot(x_ref[...], w_ref[...], preferred_element_type=jnp.float32)
        y_ref[...] = xw * dis

    return pl.pallas_call(
        body,
        grid=(GRID,),
        in_specs=[
            pl.BlockSpec((BLK, IN_DIM), lambda i: (i, 0)),
            pl.BlockSpec((IN_DIM, HID), lambda i: (0, 0)),
            pl.BlockSpec((BLK, DEG_W), lambda i: (i, 0)),
            pl.BlockSpec((BLK, DEG_W), lambda i: (i + GRID, 0)),
        ],
        out_specs=pl.BlockSpec((BLK, HID), lambda i: (i, 0)),
        out_shape=jax.ShapeDtypeStruct((N, HID), jnp.float32),
    )(x, W0, degparts, degparts)


def _tc_mid(zp, y, degparts, b, Wn):
    def body(z0_ref, z1_ref, y_ref, d0_ref, d1_ref, b_ref, w_ref, o_ref):
        dis = _dis_block(d0_ref, d1_ref)
        z = z0_ref[...] + z1_ref[...] + y_ref[...]
        h = jnp.maximum(z * dis + b_ref[...], 0.0)
        o_ref[...] = jnp.dot(h, w_ref[...], preferred_element_type=jnp.float32) * dis

    return pl.pallas_call(
        body,
        grid=(GRID,),
        in_specs=[
            pl.BlockSpec((BLK, HID), lambda i: (i, 0)),
            pl.BlockSpec((BLK, HID), lambda i: (i + GRID, 0)),
            pl.BlockSpec((BLK, HID), lambda i: (i, 0)),
            pl.BlockSpec((BLK, DEG_W), lambda i: (i, 0)),
            pl.BlockSpec((BLK, DEG_W), lambda i: (i + GRID, 0)),
            pl.BlockSpec((1, HID), lambda i: (0, 0)),
            pl.BlockSpec((HID, HID), lambda i: (0, 0)),
        ],
        out_specs=pl.BlockSpec((BLK, HID), lambda i: (i, 0)),
        out_shape=jax.ShapeDtypeStruct((N, HID), jnp.float32),
    )(zp, zp, y, degparts, degparts, b, Wn)


def _tc_final(zp, y, degparts, b):
    def body(z0_ref, z1_ref, y_ref, d0_ref, d1_ref, b_ref, o_ref):
        dis = _dis_block(d0_ref, d1_ref)
        z = z0_ref[...] + z1_ref[...] + y_ref[...]
        o_ref[...] = z * dis + b_ref[...]

    return pl.pallas_call(
        body,
        grid=(GRID,),
        in_specs=[
            pl.BlockSpec((BLK, HID), lambda i: (i, 0)),
            pl.BlockSpec((BLK, HID), lambda i: (i + GRID, 0)),
            pl.BlockSpec((BLK, HID), lambda i: (i, 0)),
            pl.BlockSpec((BLK, DEG_W), lambda i: (i, 0)),
            pl.BlockSpec((BLK, DEG_W), lambda i: (i + GRID, 0)),
            pl.BlockSpec((1, HID), lambda i: (0, 0)),
        ],
        out_specs=pl.BlockSpec((BLK, HID), lambda i: (i, 0)),
        out_shape=jax.ShapeDtypeStruct((N, HID), jnp.float32),
    )(zp, zp, y, degparts, degparts, b)


def kernel(x, edge_index, W0, b0, W1, b1, W2, b2):
    ei = edge_index.astype(jnp.int32)
    if EPTP > EPT:
        pad_src = jnp.zeros((NW, EPTP - EPT), jnp.int32)
        pad_dst = jnp.broadcast_to(
            N + (jnp.arange(EPTP - EPT, dtype=jnp.int32) % 16),
            (NW, EPTP - EPT))
        src3d = jnp.concatenate([ei[0].reshape(NW, EPT), pad_src], 1)
        dst3d = jnp.concatenate([ei[1].reshape(NW, EPT), pad_dst], 1)
    else:
        src3d, dst3d = ei[0], ei[1]
    src3d = src3d.reshape(NW, CR, CW)
    dst3d = dst3d.reshape(NW, CR, CW)

    degparts = _sc_degree(dst3d).reshape(NW * RPT, DEG_W)
    y = _tc_first(x, W0, degparts)
    zp = _sc_scatter(src3d, dst3d, y).reshape(NW * RPT, HID)
    y = _tc_mid(zp, y, degparts, b0.reshape(1, -1), W1)
    zp = _sc_scatter(src3d, dst3d, y).reshape(NW * RPT, HID)
    y = _tc_mid(zp, y, degparts, b1.reshape(1, -1), W2)
    zp = _sc_scatter(src3d, dst3d, y).reshape(NW * RPT, HID)
    out = _tc_final(zp, y, degparts, b2.reshape(1, -1))
    return out
